# baseline (speedup 1.0000x reference)
import functools

import jax
import jax.numpy as jnp
from jax.experimental import pallas as pl
from jax.experimental.pallas import tpu as pltpu

_POOL = 8


def _pool_chunk(x, rows_in, cols_in):
    io = jax.lax.broadcasted_iota
    sub = 256
    ph = (io(jnp.int32, (sub // _POOL, sub), 1) // _POOL
          == io(jnp.int32, (sub // _POOL, sub), 0)).astype(jnp.float32)
    xh = jnp.concatenate(
        [jnp.dot(ph, x[k * sub:(k + 1) * sub],
                 preferred_element_type=jnp.float32)
         for k in range(rows_in // sub)], axis=0)
    pw = (io(jnp.int32, (cols_in, cols_in // _POOL), 0) // _POOL
          == io(jnp.int32, (cols_in, cols_in // _POOL), 1)).astype(jnp.float32)
    return jnp.dot(xh, pw, preferred_element_type=jnp.float32)


def _search_step(bits, res, start, n_iter, num):
    def body(i, r):
        bitpos = jnp.int32(30) - (start + i)
        valid = bitpos >= 0
        cand = r | (jnp.int32(1) << jnp.maximum(bitpos, 0))
        cnt = jnp.sum((bits >= cand).astype(jnp.int32),
                      axis=(1, 2), keepdims=True)
        take = jnp.logical_and(valid, cnt >= num)
        return jnp.where(take, cand, r)

    return jax.lax.fori_loop(0, n_iter, body, res)


def _chs_kernel(gt_ref, gt2_ref, om0_ref, om1_ref, w_ref, out_ref,
                bits_ref, delta_ref, thr_ref, acc_ref, *,
                rows_in, cols_in, rows_out, cols_out, n_chunks, num,
                n_rows):
    b = pl.program_id(0)
    j = pl.program_id(1)
    s = b * n_chunks + j

    @pl.when(s == 0)
    def _init():
        acc_ref[0] = 0.0

    dmap = _pool_chunk(gt_ref[0, 0], rows_in, cols_in)
    dmap = dmap + 0.0 * gt2_ref[0, 0, :rows_out, :cols_out]
    om0 = om0_ref[0]
    om1 = om1_ref[0]
    w = w_ref[0]
    d_base = om0 - dmap
    err = jnp.abs(d_base)
    bits_ref[b, pl.ds(j * rows_out, rows_out)] = (
        jax.lax.bitcast_convert_type(err, jnp.int32))
    d_comb = om0 - (w * om1 + (1.0 - w) * dmap)
    base = d_base * d_base
    delta_ref[b, pl.ds(j * rows_out, rows_out)] = d_comb * d_comb - base
    acc_ref[0] += jnp.sum(base)

    steps_per_pair = 2 * n_chunks
    it_per_step = 32 // steps_per_pair
    sp = s - steps_per_pair
    p = sp // steps_per_pair
    k = sp % steps_per_pair
    searching = (sp >= 0) & (p < n_rows // 2 - 1)

    @pl.when(searching & (k == 0))
    def _start_pair():
        thr_ref[...] = jnp.zeros((2, 1, 1), jnp.int32)

    @pl.when(searching)
    def _advance_pair():
        bits = bits_ref[pl.ds(2 * p, 2)]
        res = _search_step(bits, thr_ref[...], k * it_per_step,
                           it_per_step, num)
        thr_ref[...] = res

        @pl.when(k == steps_per_pair - 1)
        def _finish_pair():
            corr = jnp.where(bits >= res, delta_ref[pl.ds(2 * p, 2)], 0.0)
            acc_ref[0] += jnp.sum(corr)

    @pl.when(s == n_rows * n_chunks - 1)
    def _tail():
        base_row = n_rows - 2
        bits = bits_ref[pl.ds(base_row, 2)]
        res = _search_step(bits, jnp.zeros((2, 1, 1), jnp.int32), 0, 31,
                           num)
        corr = jnp.where(bits >= res, delta_ref[pl.ds(base_row, 2)], 0.0)
        out_ref[...] = jnp.full((1, 1), acc_ref[0] + jnp.sum(corr),
                                jnp.float32)


def kernel(output_map_0, output_map_1, gt_density, process):
    b, c, h, w = output_map_0.shape
    B, C, H, W = gt_density.shape
    num = int(h * w * 0.1)

    rows_in = 1024
    rows_out = rows_in // _POOL
    n_chunks = H // rows_in

    om0 = output_map_0.reshape(B, h, w)
    om1 = output_map_1.reshape(B, h, w)
    wmat = jnp.broadcast_to(jnp.asarray(process, jnp.float32), (1, 1, 1))

    loss = pl.pallas_call(
        functools.partial(_chs_kernel, rows_in=rows_in, cols_in=W,
                          rows_out=rows_out, cols_out=w,
                          n_chunks=n_chunks, num=num, n_rows=B),
        grid=(B, n_chunks),
        in_specs=[
            pl.BlockSpec((1, 1, rows_in, W), lambda bi, j: (bi, 0, j, 0)),
            pl.BlockSpec((1, 1, rows_in, W), lambda bi, j: (bi, 0, j, 0)),
            pl.BlockSpec((1, rows_out, w), lambda bi, j: (bi, j, 0)),
            pl.BlockSpec((1, rows_out, w), lambda bi, j: (bi, j, 0)),
            pl.BlockSpec((1, 1, 1), lambda bi, j: (0, 0, 0)),
        ],
        out_specs=pl.BlockSpec((1, 1), lambda bi, j: (0, 0)),
        out_shape=jax.ShapeDtypeStruct((1, 1), jnp.float32),
        scratch_shapes=[
            pltpu.VMEM((B, h, w), jnp.int32),
            pltpu.VMEM((B, h, w), jnp.float32),
            pltpu.VMEM((2, 1, 1), jnp.int32),
            pltpu.SMEM((1,), jnp.float32),
        ],
    )(gt_density, gt_density, om0, om1, wmat)
    return loss[0, 0]


# speedup vs baseline: 1.5988x; 1.5988x over previous
import functools

import jax
import jax.numpy as jnp
from jax.experimental import pallas as pl
from jax.experimental.pallas import tpu as pltpu

_POOL = 8


def _pool_chunk(x, rows_in, cols_in):
    io = jax.lax.broadcasted_iota
    sub = 256
    ph = (io(jnp.int32, (sub // _POOL, sub), 1) // _POOL
          == io(jnp.int32, (sub // _POOL, sub), 0)).astype(jnp.float32)
    xh = jnp.concatenate(
        [jnp.dot(ph, x[k * sub:(k + 1) * sub],
                 preferred_element_type=jnp.float32)
         for k in range(rows_in // sub)], axis=0)
    pw = (io(jnp.int32, (cols_in, cols_in // _POOL), 0) // _POOL
          == io(jnp.int32, (cols_in, cols_in // _POOL), 1)).astype(jnp.float32)
    return jnp.dot(xh, pw, preferred_element_type=jnp.float32)


def _search_step(bits, res, start, n_iter, num):
    def body(i, r):
        bitpos = jnp.int32(30) - (start + i)
        valid = bitpos >= 0
        cand = r | (jnp.int32(1) << jnp.maximum(bitpos, 0))
        cnt = jnp.sum((bits >= cand).astype(jnp.int32),
                      axis=(1, 2), keepdims=True)
        take = jnp.logical_and(valid, cnt >= num)
        return jnp.where(take, cand, r)

    return jax.lax.fori_loop(0, n_iter, body, res)


def _chs_kernel(gta_ref, gtb_ref, om0_ref, om1_ref, w_ref, out_ref,
                bits_ref, delta_ref, thr_ref, acc_ref, *,
                rows_in, cols_in, rows_out, cols_out, n_chunks, num,
                n_pairs):
    q = pl.program_id(0)
    j = pl.program_id(1)

    @pl.when((q == 0) & (j == 0))
    def _init():
        acc_ref[0] = 0.0

    for t, gt_ref in ((0, gta_ref), (1, gtb_ref)):
        dmap = _pool_chunk(gt_ref[0, 0], rows_in, cols_in)
        om0 = om0_ref[t]
        om1 = om1_ref[t]
        w = w_ref[0]
        d_base = om0 - dmap
        err = jnp.abs(d_base)
        bits_ref[2 * q + t, pl.ds(j * rows_out, rows_out)] = (
            jax.lax.bitcast_convert_type(err, jnp.int32))
        d_comb = om0 - (w * om1 + (1.0 - w) * dmap)
        base = d_base * d_base
        delta_ref[2 * q + t, pl.ds(j * rows_out, rows_out)] = (
            d_comb * d_comb - base)
        acc_ref[0] += jnp.sum(base)

    it_per_step = 32 // n_chunks
    p = q - 1
    searching = p >= 0

    @pl.when(searching & (j == 0))
    def _start_pair():
        thr_ref[...] = jnp.zeros((2, 1, 1), jnp.int32)

    @pl.when(searching)
    def _advance_pair():
        bits = bits_ref[pl.ds(2 * p, 2)]
        res = _search_step(bits, thr_ref[...], j * it_per_step,
                           it_per_step, num)
        thr_ref[...] = res

        @pl.when(j == n_chunks - 1)
        def _finish_pair():
            corr = jnp.where(bits >= res, delta_ref[pl.ds(2 * p, 2)], 0.0)
            acc_ref[0] += jnp.sum(corr)

    @pl.when((q == n_pairs - 1) & (j == n_chunks - 1))
    def _tail():
        base_row = 2 * (n_pairs - 1)
        bits = bits_ref[pl.ds(base_row, 2)]
        res = _search_step(bits, jnp.zeros((2, 1, 1), jnp.int32), 0, 31,
                           num)
        corr = jnp.where(bits >= res, delta_ref[pl.ds(base_row, 2)], 0.0)
        out_ref[...] = jnp.full((1, 1), acc_ref[0] + jnp.sum(corr),
                                jnp.float32)


def kernel(output_map_0, output_map_1, gt_density, process):
    b, c, h, w = output_map_0.shape
    B, C, H, W = gt_density.shape
    num = int(h * w * 0.1)

    rows_in = 1024
    rows_out = rows_in // _POOL
    n_chunks = H // rows_in
    n_pairs = B // 2

    om0 = output_map_0.reshape(B, h, w)
    om1 = output_map_1.reshape(B, h, w)
    wmat = jnp.broadcast_to(jnp.asarray(process, jnp.float32), (1, 1, 1))

    loss = pl.pallas_call(
        functools.partial(_chs_kernel, rows_in=rows_in, cols_in=W,
                          rows_out=rows_out, cols_out=w,
                          n_chunks=n_chunks, num=num, n_pairs=n_pairs),
        grid=(n_pairs, n_chunks),
        in_specs=[
            pl.BlockSpec((1, 1, rows_in, W), lambda q, j: (2 * q, 0, j, 0)),
            pl.BlockSpec((1, 1, rows_in, W),
                         lambda q, j: (2 * q + 1, 0, j, 0)),
            pl.BlockSpec((2, rows_out, w), lambda q, j: (q, j, 0)),
            pl.BlockSpec((2, rows_out, w), lambda q, j: (q, j, 0)),
            pl.BlockSpec((1, 1, 1), lambda q, j: (0, 0, 0)),
        ],
        out_specs=pl.BlockSpec((1, 1), lambda q, j: (0, 0)),
        out_shape=jax.ShapeDtypeStruct((1, 1), jnp.float32),
        scratch_shapes=[
            pltpu.VMEM((B, h, w), jnp.int32),
            pltpu.VMEM((B, h, w), jnp.float32),
            pltpu.VMEM((2, 1, 1), jnp.int32),
            pltpu.SMEM((1,), jnp.float32),
        ],
    )(gt_density, gt_density, om0, om1, wmat)
    return loss[0, 0]
